# baseline (speedup 1.0000x reference)
import functools

import jax
import jax.numpy as jnp
from jax import lax
from jax.experimental import pallas as pl
from jax.experimental.pallas import tpu as pltpu
from jax.experimental.pallas import tpu_sc as plsc

_VOCAB = 1000000
_D = 64
_SCALE = 8.0
_B = 16384 * 50
_NW = 32
_CHUNK = 128
_ROWS_PER_W = _B // _NW
_CHUNKS_PER_W = _ROWS_PER_W // _CHUNK


@functools.partial(
    pl.kernel,
    mesh=plsc.VectorSubcoreMesh(core_axis_name="c", subcore_axis_name="s"),
    out_type=jax.ShapeDtypeStruct((_B, _D), jnp.float32),
    scratch_types=[
        pltpu.VMEM((_CHUNKS_PER_W, _CHUNK), jnp.int32),
        pltpu.VMEM((_CHUNK, _D), jnp.float32),
        pltpu.VMEM((_CHUNK,), jnp.float32),
        pltpu.SemaphoreType.DMA,
    ],
    compiler_params=pltpu.CompilerParams(
        needs_layout_passes=False, use_tc_tiling_on_sc=False
    ),
)
def _emb_lookup(idx_hbm, table_hbm, out_hbm, idx_v, rows_v, scales_v, sem):
    nc = 2
    wid = lax.axis_index("s") * nc + lax.axis_index("c")

    pltpu.sync_copy(idx_hbm.at[pl.ds(wid * _CHUNKS_PER_W, _CHUNKS_PER_W)], idx_v)

    def chunk_body(g, _):
        pltpu.async_copy(table_hbm.at[idx_v.at[g]], rows_v, sem).wait()

        def sgrp_body(h, _):
            iv = idx_v[g, pl.ds(h * 16, 16)]
            scales_v[pl.ds(h * 16, 16)] = jnp.where(iv == 0, 0.0, _SCALE)
            return 0

        lax.fori_loop(0, _CHUNK // 16, sgrp_body, 0)

        def row_body(r, _):
            f = plsc.load_gather(scales_v, [jnp.full((16,), r, jnp.int32)])
            for c in range(_D // 16):
                rows_v[r, pl.ds(c * 16, 16)] = rows_v[r, pl.ds(c * 16, 16)] * f
            return 0

        lax.fori_loop(0, _CHUNK, row_body, 0)

        base = wid * _ROWS_PER_W + g * _CHUNK
        pltpu.sync_copy(rows_v, out_hbm.at[pl.ds(base, _CHUNK)])
        return 0

    lax.fori_loop(0, _CHUNKS_PER_W, chunk_body, 0)


def kernel(x, table):
    idx = x.reshape(_B // _CHUNK, _CHUNK).astype(jnp.int32)
    out = _emb_lookup(idx, table)
    return out.reshape(x.shape[0], x.shape[1], _D)


# speedup vs baseline: 1.1751x; 1.1751x over previous
import functools

import jax
import jax.numpy as jnp
from jax import lax
from jax.experimental import pallas as pl
from jax.experimental.pallas import tpu as pltpu
from jax.experimental.pallas import tpu_sc as plsc

_D = 64
_SCALE = 8.0
_B = 16384 * 50
_NW = 32
_IDXW = 128
_CHUNK = 256
_ROWS_PER_W = _B // _NW
_IDXROWS_PER_W = _ROWS_PER_W // _IDXW
_CHUNKS_PER_W = _ROWS_PER_W // _CHUNK
_PAIRS = _CHUNKS_PER_W // 2


@functools.partial(
    pl.kernel,
    mesh=plsc.VectorSubcoreMesh(core_axis_name="c", subcore_axis_name="s"),
    out_type=jax.ShapeDtypeStruct((_B, _D), jnp.float32),
    scratch_types=[
        pltpu.VMEM((_IDXROWS_PER_W, _IDXW), jnp.int32),
        pltpu.VMEM((_CHUNK, _D), jnp.float32),
        pltpu.VMEM((_CHUNK, _D), jnp.float32),
        pltpu.VMEM((_CHUNK,), jnp.float32),
        pltpu.SemaphoreType.DMA,
        pltpu.SemaphoreType.DMA,
        pltpu.SemaphoreType.DMA,
        pltpu.SemaphoreType.DMA,
    ],
    compiler_params=pltpu.CompilerParams(
        needs_layout_passes=False, use_tc_tiling_on_sc=False
    ),
)
def _emb_lookup(idx_hbm, table_hbm, out_hbm, idx_v, rows0, rows1,
                scales_v, gsem0, gsem1, wsem0, wsem1):
    nc = 2
    wid = lax.axis_index("s") * nc + lax.axis_index("c")
    row_base = wid * _ROWS_PER_W

    pltpu.sync_copy(
        idx_hbm.at[pl.ds(wid * _IDXROWS_PER_W, _IDXROWS_PER_W)], idx_v
    )

    def start_gather(g, buf, sem):
        pltpu.async_copy(
            table_hbm.at[idx_v.at[2 * g]], buf.at[pl.ds(0, _IDXW)], sem
        )
        pltpu.async_copy(
            table_hbm.at[idx_v.at[2 * g + 1]], buf.at[pl.ds(_IDXW, _IDXW)], sem
        )

    def wait_gather(buf, sem):
        pltpu.make_async_copy(
            table_hbm.at[idx_v.at[0]], buf.at[pl.ds(0, _IDXW)], sem
        ).wait()
        pltpu.make_async_copy(
            table_hbm.at[idx_v.at[0]], buf.at[pl.ds(_IDXW, _IDXW)], sem
        ).wait()

    def start_writeout(g, buf, sem):
        pltpu.async_copy(
            buf, out_hbm.at[pl.ds(row_base + g * _CHUNK, _CHUNK)], sem
        )

    def wait_writeout(buf, sem):
        pltpu.make_async_copy(
            buf, out_hbm.at[pl.ds(row_base, _CHUNK)], sem
        ).wait()

    def compute(g, buf):
        def sgrp_body(h, _):
            iv = idx_v[2 * g + h // 8, pl.ds((h % 8) * 16, 16)]
            scales_v[pl.ds(h * 16, 16)] = jnp.where(iv == 0, 0.0, _SCALE)
            return 0

        lax.fori_loop(0, _CHUNK // 16, sgrp_body, 0)

        def row_body(q, _):
            for u in range(4):
                r = q * 4 + u
                f = plsc.load_gather(
                    scales_v, [jnp.full((16,), r, jnp.int32)]
                )
                for c in range(_D // 16):
                    buf[r, pl.ds(c * 16, 16)] = buf[r, pl.ds(c * 16, 16)] * f
            return 0

        lax.fori_loop(0, _CHUNK // 4, row_body, 0)

    start_gather(0, rows0, gsem0)

    def pair_body(i, _):
        a = 2 * i
        b = a + 1

        @pl.when(i > 0)
        def _():
            wait_writeout(rows1, wsem1)

        start_gather(b, rows1, gsem1)

        wait_gather(rows0, gsem0)
        compute(a, rows0)
        start_writeout(a, rows0, wsem0)

        wait_gather(rows1, gsem1)
        wait_writeout(rows0, wsem0)

        @pl.when(i < _PAIRS - 1)
        def _():
            start_gather(a + 2, rows0, gsem0)

        compute(b, rows1)
        start_writeout(b, rows1, wsem1)
        return 0

    lax.fori_loop(0, _PAIRS, pair_body, 0)
    wait_writeout(rows1, wsem1)


def kernel(x, table):
    idx = x.reshape(_B // _IDXW, _IDXW).astype(jnp.int32)
    out = _emb_lookup(idx, table)
    return out.reshape(x.shape[0], x.shape[1], _D)
